# baseline (speedup 1.0000x reference)
import functools

import jax
import jax.numpy as jnp
from jax import lax
from jax.experimental import pallas as pl
from jax.experimental.pallas import tpu as pltpu
from jax.experimental.pallas import tpu_sc as plsc

B, N, D_s, D_id, H, Dq, N_in, V = 128, 1024, 256, 64, 4, 64, 256, 32768

NB = 128
VB = 4096

SC_NC, SC_NS = 2, 16
SC_NW = 16
SC_BPW = B // SC_NW


def _gather_sc_body(table_ref, idx_ref, out_ref, idx_v, rows_v, sem):
    wid = lax.axis_index("s") * SC_NC + lax.axis_index("c")

    @pl.when(wid < SC_NW)
    def _():
        base = wid * SC_BPW
        pltpu.sync_copy(idx_ref.at[pl.ds(base, SC_BPW)], idx_v)
        pltpu.async_copy(table_ref.at[idx_v], rows_v, sem).wait()
        pltpu.sync_copy(rows_v, out_ref.at[pl.ds(base, SC_BPW)])


def _gather_h(token_id, tok_emb):
    return pl.kernel(
        _gather_sc_body,
        out_type=jax.ShapeDtypeStruct((B, D_s), jnp.float32),
        mesh=plsc.VectorSubcoreMesh(core_axis_name="c", subcore_axis_name="s"),
        scratch_types=[
            pltpu.VMEM((SC_BPW,), jnp.int32),
            pltpu.VMEM((SC_BPW, D_s), jnp.float32),
            pltpu.SemaphoreType.DMA,
        ],
    )(tok_emb, token_id)


def _prep_body(h_ref, wqt_ref, colid_ref, wkin_ref, wvin_ref,
               wdec_ref, bdec_ref, ebias_ref, wkout_ref, mq_ref, pos_ref,
               alpha_ref, count_ref, v_ref, vdot_ref, mk_ref):
    h = h_ref[...]

    col_iota = jax.lax.broadcasted_iota(jnp.int32, (N_in, N), 1)
    P = (col_iota == pos_ref[...]).astype(jnp.float32)
    in_ids = jax.lax.dot_general(P, colid_ref[...],
                                 (((1,), (0,)), ((), ())))
    keys = jax.lax.dot_general(in_ids, wkin_ref[...],
                               (((1,), (0,)), ((), ())))
    j_iota = jax.lax.broadcasted_iota(jnp.int32, (B, N_in), 1)
    counts_in = jnp.zeros((B, N_in), jnp.float32)
    for hd in range(H):
        wqt_h = wqt_ref[hd * Dq:(hd + 1) * Dq, :]
        a_h = jax.lax.dot_general(wqt_h, keys,
                                  (((0,), (1,)), ((), ())))
        sc_h = jax.lax.dot_general(h, a_h, (((1,), (0,)), ((), ())))
        sc_h = sc_h * (1.0 / 8.0) + ebias_ref[hd:hd + 1, :]
        mx = jnp.max(sc_h, axis=1, keepdims=True)
        idx = jnp.min(jnp.where(sc_h == mx, j_iota, N_in),
                      axis=1, keepdims=True)
        counts_in = counts_in + (j_iota == idx).astype(jnp.float32)
    count_ref[...] = jax.lax.dot_general(counts_in, P,
                                         (((1,), (0,)), ((), ())))
    v = jax.lax.dot_general(h, wvin_ref[...], (((1,), (0,)), ((), ())))
    v_ref[...] = v
    a = jax.lax.dot_general(colid_ref[...], wdec_ref[...],
                            (((1,), (0,)), ((), ())))
    alpha_ref[...] = jax.nn.sigmoid(a + bdec_ref[0, 0])
    mk = jax.lax.dot_general(mq_ref[...], wkout_ref[...],
                             (((1,), (1,)), ((), ())))
    mk_ref[...] = mk
    vdot_ref[...] = jax.lax.dot_general(v, mk, (((1,), (1,)), ((), ())))


def _prep(h, WqT, col_id, Wk_in, Wv_in, w_decay, b_decay,
          input_E_bias, Wk_out, motor_query, input_positions):
    out_shapes = (
        jax.ShapeDtypeStruct((N, 1), jnp.float32),
        jax.ShapeDtypeStruct((B, N), jnp.float32),
        jax.ShapeDtypeStruct((B, D_s), jnp.float32),
        jax.ShapeDtypeStruct((B, 1), jnp.float32),
        jax.ShapeDtypeStruct((1, D_s), jnp.float32),
    )
    return pl.pallas_call(
        _prep_body,
        out_shape=out_shapes,
    )(h, WqT, col_id, Wk_in, Wv_in, w_decay,
      b_decay.reshape(1, 1), input_E_bias, Wk_out,
      motor_query.reshape(1, D_s), input_positions.reshape(N_in, 1))


def _stream_body(s_ref, alpha_ref, count_ref, v_ref, vdot_ref, mk_ref,
                 wtd_ref, m_sc, z_sc, cv_sc, acc_sc):
    i = pl.program_id(0)

    @pl.when(i == 0)
    def _init():
        m_sc[...] = jnp.full((B, 1), -1e30, jnp.float32)
        z_sc[...] = jnp.zeros((B, 1), jnp.float32)
        cv_sc[...] = jnp.zeros((B, 1), jnp.float32)
        acc_sc[...] = jnp.zeros((B, D_s), jnp.float32)

    s_blk = s_ref[...]
    sdot = jax.lax.dot_general(
        s_blk.reshape(B * NB, D_s), mk_ref[...],
        (((1,), (1,)), ((), ()))).reshape(B, NB, 1)[:, :, 0]
    alpha = alpha_ref[...]
    cnt = count_ref[...]
    logit = (alpha * sdot + cnt * vdot_ref[...]) * (1.0 / 16.0)
    m_old = m_sc[...]
    m_new = jnp.maximum(m_old, jnp.max(logit, axis=1, keepdims=True))
    corr = jnp.exp(m_old - m_new)
    p = jnp.exp(logit - m_new)
    m_sc[...] = m_new
    z_sc[...] = z_sc[...] * corr + jnp.sum(p, axis=1, keepdims=True)
    cv_sc[...] = cv_sc[...] * corr + jnp.sum(p * cnt, axis=1, keepdims=True)
    pa = p * alpha
    contrib = jax.lax.dot_general(pa, s_blk,
                                  (((1,), (1,)), ((0,), (0,))))
    acc_sc[...] = acc_sc[...] * corr + contrib

    @pl.when(i == (N // NB) - 1)
    def _fin():
        wtd_ref[...] = (acc_sc[...] + cv_sc[...] * v_ref[...]) / z_sc[...]


def _stream(s, alpha2, count, v, vdot, mk):
    return pl.pallas_call(
        _stream_body,
        grid=(N // NB,),
        in_specs=[
            pl.BlockSpec((B, NB, D_s), lambda i: (0, i, 0)),
            pl.BlockSpec((1, NB), lambda i: (0, i)),
            pl.BlockSpec((B, NB), lambda i: (0, i)),
            pl.BlockSpec((B, D_s), lambda i: (0, 0)),
            pl.BlockSpec((B, 1), lambda i: (0, 0)),
            pl.BlockSpec((1, D_s), lambda i: (0, 0)),
        ],
        out_specs=pl.BlockSpec((B, D_s), lambda i: (0, 0)),
        out_shape=jax.ShapeDtypeStruct((B, D_s), jnp.float32),
        scratch_shapes=[
            pltpu.VMEM((B, 1), jnp.float32),
            pltpu.VMEM((B, 1), jnp.float32),
            pltpu.VMEM((B, 1), jnp.float32),
            pltpu.VMEM((B, D_s), jnp.float32),
        ],
    )(s, alpha2, count, v, vdot, mk)


def _logits_body(wtd_ref, wvout_ref, emb_ref, out_ref, motor_sc):
    @pl.when(pl.program_id(0) == 0)
    def _motor():
        motor = jax.lax.dot_general(wtd_ref[...], wvout_ref[...],
                                    (((1,), (0,)), ((), ())))
        ms = jnp.mean(motor * motor, axis=-1, keepdims=True)
        motor_sc[...] = motor * jax.lax.rsqrt(ms + 1e-6)

    out_ref[...] = jax.lax.dot_general(motor_sc[...], emb_ref[...],
                                       (((1,), (1,)), ((), ())))


def _logits(weighted, Wv_out, tok_emb):
    return pl.pallas_call(
        _logits_body,
        grid=(V // VB,),
        in_specs=[
            pl.BlockSpec((B, D_s), lambda i: (0, 0)),
            pl.BlockSpec((D_s, D_s), lambda i: (0, 0)),
            pl.BlockSpec((VB, D_s), lambda i: (i, 0)),
        ],
        out_specs=pl.BlockSpec((B, VB), lambda i: (0, i)),
        out_shape=jax.ShapeDtypeStruct((B, V), jnp.float32),
        scratch_shapes=[pltpu.VMEM((B, D_s), jnp.float32)],
    )(weighted, Wv_out, tok_emb)


def kernel(token_id, s, tok_emb, Wq, col_id, Wk_in, Wv_in, w_decay, b_decay,
           input_E_bias, Wk_out, Wv_out, motor_query, input_positions):
    h = _gather_h(token_id.astype(jnp.int32), tok_emb)
    alpha, count, v, vdot, mk = _prep(
        h, Wq.T, col_id, Wk_in, Wv_in, w_decay, b_decay,
        input_E_bias, Wk_out, motor_query, input_positions)
    weighted = _stream(s, alpha.reshape(1, N), count, v, vdot, mk)
    return _logits(weighted, Wv_out, tok_emb)


# speedup vs baseline: 1.1457x; 1.1457x over previous
import functools

import jax
import jax.numpy as jnp
from jax import lax
from jax.experimental import pallas as pl
from jax.experimental.pallas import tpu as pltpu
from jax.experimental.pallas import tpu_sc as plsc

B, N, D_s, D_id, H, Dq, N_in, V = 128, 1024, 256, 64, 4, 64, 256, 32768

NB = 128
VB = 4096

SC_NC, SC_NS = 2, 16
SC_NW = 16
SC_BPW = B // SC_NW


def _gather_sc_body(table_ref, idx_ref, out_ref, idx_v, rows_v, sem):
    wid = lax.axis_index("s") * SC_NC + lax.axis_index("c")

    @pl.when(wid < SC_NW)
    def _():
        base = wid * SC_BPW
        pltpu.sync_copy(idx_ref.at[pl.ds(base, SC_BPW)], idx_v)
        pltpu.async_copy(table_ref.at[idx_v], rows_v, sem).wait()
        pltpu.sync_copy(rows_v, out_ref.at[pl.ds(base, SC_BPW)])


def _gather_h(token_id, tok_emb):
    return pl.kernel(
        _gather_sc_body,
        out_type=jax.ShapeDtypeStruct((B, D_s), jnp.float32),
        mesh=plsc.VectorSubcoreMesh(core_axis_name="c", subcore_axis_name="s"),
        scratch_types=[
            pltpu.VMEM((SC_BPW,), jnp.int32),
            pltpu.VMEM((SC_BPW, D_s), jnp.float32),
            pltpu.SemaphoreType.DMA,
        ],
    )(tok_emb, token_id)


def _prep_body(tok_ref, emb_ref, wqt_ref, colid_ref, wkin_ref, wvin_ref,
               wdec_ref, bdec_ref, ebias_ref, wkout_ref, mq_ref, pos_ref,
               alpha_ref, count_ref, v_ref, vdot_ref, mk_ref,
               h_sc, sems):
    for b in range(B):
        pltpu.make_async_copy(
            emb_ref.at[pl.ds(tok_ref[b], 1), :],
            h_sc.at[pl.ds(b, 1), :], sems.at[b % 4]).start()

    col_iota = jax.lax.broadcasted_iota(jnp.int32, (N_in, N), 1)
    P = (col_iota == pos_ref[...]).astype(jnp.float32)
    in_ids = jax.lax.dot_general(P, colid_ref[...],
                                 (((1,), (0,)), ((), ())))
    keys = jax.lax.dot_general(in_ids, wkin_ref[...],
                               (((1,), (0,)), ((), ())))
    a = jax.lax.dot_general(colid_ref[...], wdec_ref[...],
                            (((1,), (0,)), ((), ())))
    alpha_ref[...] = jax.nn.sigmoid(a + bdec_ref[0, 0])
    mk = jax.lax.dot_general(mq_ref[...], wkout_ref[...],
                             (((1,), (1,)), ((), ())))
    mk_ref[...] = mk

    for b in range(B):
        pltpu.make_async_copy(
            emb_ref.at[pl.ds(tok_ref[b], 1), :],
            h_sc.at[pl.ds(b, 1), :], sems.at[b % 4]).wait()
    h = h_sc[...]
    j_iota = jax.lax.broadcasted_iota(jnp.int32, (B, N_in), 1)
    counts_in = jnp.zeros((B, N_in), jnp.float32)
    for hd in range(H):
        wqt_h = wqt_ref[hd * Dq:(hd + 1) * Dq, :]
        a_h = jax.lax.dot_general(wqt_h, keys,
                                  (((0,), (1,)), ((), ())))
        sc_h = jax.lax.dot_general(h, a_h, (((1,), (0,)), ((), ())))
        sc_h = sc_h * (1.0 / 8.0) + ebias_ref[hd:hd + 1, :]
        mx = jnp.max(sc_h, axis=1, keepdims=True)
        idx = jnp.min(jnp.where(sc_h == mx, j_iota, N_in),
                      axis=1, keepdims=True)
        counts_in = counts_in + (j_iota == idx).astype(jnp.float32)
    count_ref[...] = jax.lax.dot_general(counts_in, P,
                                         (((1,), (0,)), ((), ())))
    v = jax.lax.dot_general(h, wvin_ref[...], (((1,), (0,)), ((), ())))
    v_ref[...] = v
    vdot_ref[...] = jax.lax.dot_general(v, mk, (((1,), (1,)), ((), ())))


def _prep(token_id, tok_emb, WqT, col_id, Wk_in, Wv_in, w_decay, b_decay,
          input_E_bias, Wk_out, motor_query, input_positions):
    out_shapes = (
        jax.ShapeDtypeStruct((N, 1), jnp.float32),
        jax.ShapeDtypeStruct((B, N), jnp.float32),
        jax.ShapeDtypeStruct((B, D_s), jnp.float32),
        jax.ShapeDtypeStruct((B, 1), jnp.float32),
        jax.ShapeDtypeStruct((1, D_s), jnp.float32),
    )
    return pl.pallas_call(
        _prep_body,
        grid_spec=pltpu.PrefetchScalarGridSpec(
            num_scalar_prefetch=1,
            grid=(1,),
            in_specs=[
                pl.BlockSpec(memory_space=pltpu.MemorySpace.HBM),
                pl.BlockSpec((H * Dq, D_s), lambda i, tok: (0, 0)),
                pl.BlockSpec((N, D_id), lambda i, tok: (0, 0)),
                pl.BlockSpec((D_id, Dq), lambda i, tok: (0, 0)),
                pl.BlockSpec((D_s, D_s), lambda i, tok: (0, 0)),
                pl.BlockSpec((D_id, 1), lambda i, tok: (0, 0)),
                pl.BlockSpec((1, 1), lambda i, tok: (0, 0)),
                pl.BlockSpec((H, N_in), lambda i, tok: (0, 0)),
                pl.BlockSpec((D_s, D_s), lambda i, tok: (0, 0)),
                pl.BlockSpec((1, D_s), lambda i, tok: (0, 0)),
                pl.BlockSpec((N_in, 1), lambda i, tok: (0, 0)),
            ],
            out_specs=[
                pl.BlockSpec((N, 1), lambda i, tok: (0, 0)),
                pl.BlockSpec((B, N), lambda i, tok: (0, 0)),
                pl.BlockSpec((B, D_s), lambda i, tok: (0, 0)),
                pl.BlockSpec((B, 1), lambda i, tok: (0, 0)),
                pl.BlockSpec((1, D_s), lambda i, tok: (0, 0)),
            ],
            scratch_shapes=[
                pltpu.VMEM((B, D_s), jnp.float32),
                pltpu.SemaphoreType.DMA((4,)),
            ],
        ),
        out_shape=out_shapes,
    )(token_id, tok_emb, WqT, col_id, Wk_in, Wv_in, w_decay,
      b_decay.reshape(1, 1), input_E_bias, Wk_out,
      motor_query.reshape(1, D_s), input_positions.reshape(N_in, 1))


def _stream_body(s_ref, alpha_ref, count_ref, v_ref, vdot_ref, mk_ref,
                 wtd_ref, m_sc, z_sc, cv_sc, acc_sc):
    i = pl.program_id(0)

    @pl.when(i == 0)
    def _init():
        m_sc[...] = jnp.full((B, 1), -1e30, jnp.float32)
        z_sc[...] = jnp.zeros((B, 1), jnp.float32)
        cv_sc[...] = jnp.zeros((B, 1), jnp.float32)
        acc_sc[...] = jnp.zeros((B, D_s), jnp.float32)

    s_blk = s_ref[...]
    sdot = jax.lax.dot_general(
        s_blk.reshape(B * NB, D_s), mk_ref[...],
        (((1,), (1,)), ((), ()))).reshape(B, NB, 1)[:, :, 0]
    alpha = alpha_ref[...]
    cnt = count_ref[...]
    logit = (alpha * sdot + cnt * vdot_ref[...]) * (1.0 / 16.0)
    m_old = m_sc[...]
    m_new = jnp.maximum(m_old, jnp.max(logit, axis=1, keepdims=True))
    corr = jnp.exp(m_old - m_new)
    p = jnp.exp(logit - m_new)
    m_sc[...] = m_new
    z_sc[...] = z_sc[...] * corr + jnp.sum(p, axis=1, keepdims=True)
    cv_sc[...] = cv_sc[...] * corr + jnp.sum(p * cnt, axis=1, keepdims=True)
    pa = p * alpha
    contrib = jax.lax.dot_general(pa, s_blk,
                                  (((1,), (1,)), ((0,), (0,))))
    acc_sc[...] = acc_sc[...] * corr + contrib

    @pl.when(i == (N // NB) - 1)
    def _fin():
        wtd_ref[...] = (acc_sc[...] + cv_sc[...] * v_ref[...]) / z_sc[...]


def _stream(s, alpha2, count, v, vdot, mk):
    return pl.pallas_call(
        _stream_body,
        grid=(N // NB,),
        in_specs=[
            pl.BlockSpec((B, NB, D_s), lambda i: (0, i, 0)),
            pl.BlockSpec((1, NB), lambda i: (0, i)),
            pl.BlockSpec((B, NB), lambda i: (0, i)),
            pl.BlockSpec((B, D_s), lambda i: (0, 0)),
            pl.BlockSpec((B, 1), lambda i: (0, 0)),
            pl.BlockSpec((1, D_s), lambda i: (0, 0)),
        ],
        out_specs=pl.BlockSpec((B, D_s), lambda i: (0, 0)),
        out_shape=jax.ShapeDtypeStruct((B, D_s), jnp.float32),
        scratch_shapes=[
            pltpu.VMEM((B, 1), jnp.float32),
            pltpu.VMEM((B, 1), jnp.float32),
            pltpu.VMEM((B, 1), jnp.float32),
            pltpu.VMEM((B, D_s), jnp.float32),
        ],
    )(s, alpha2, count, v, vdot, mk)


def _logits_body(wtd_ref, wvout_ref, emb_ref, out_ref, motor_sc):
    @pl.when(pl.program_id(0) == 0)
    def _motor():
        motor = jax.lax.dot_general(wtd_ref[...], wvout_ref[...],
                                    (((1,), (0,)), ((), ())))
        ms = jnp.mean(motor * motor, axis=-1, keepdims=True)
        motor_sc[...] = motor * jax.lax.rsqrt(ms + 1e-6)

    out_ref[...] = jax.lax.dot_general(motor_sc[...], emb_ref[...],
                                       (((1,), (1,)), ((), ())))


def _logits(weighted, Wv_out, tok_emb):
    return pl.pallas_call(
        _logits_body,
        grid=(V // VB,),
        in_specs=[
            pl.BlockSpec((B, D_s), lambda i: (0, 0)),
            pl.BlockSpec((D_s, D_s), lambda i: (0, 0)),
            pl.BlockSpec((VB, D_s), lambda i: (i, 0)),
        ],
        out_specs=pl.BlockSpec((B, VB), lambda i: (0, i)),
        out_shape=jax.ShapeDtypeStruct((B, V), jnp.float32),
        scratch_shapes=[pltpu.VMEM((B, D_s), jnp.float32)],
    )(weighted, Wv_out, tok_emb)


def kernel(token_id, s, tok_emb, Wq, col_id, Wk_in, Wv_in, w_decay, b_decay,
           input_E_bias, Wk_out, Wv_out, motor_query, input_positions):
    alpha, count, v, vdot, mk = _prep(
        token_id, tok_emb, Wq.T, col_id, Wk_in, Wv_in, w_decay, b_decay,
        input_E_bias, Wk_out, motor_query, input_positions)
    weighted = _stream(s, alpha.reshape(1, N), count, v, vdot, mk)
    return _logits(weighted, Wv_out, tok_emb)


# speedup vs baseline: 1.1942x; 1.0424x over previous
import jax
import jax.numpy as jnp
from jax.experimental import pallas as pl
from jax.experimental.pallas import tpu as pltpu

B, N, D_s, D_id, H, Dq, N_in, V = 128, 1024, 256, 64, 4, 64, 256, 32768

NB = 128
VB = 4096


def _stream_body(tok_ref, emb_ref, s_ref, wqt_ref, colidt_ref, wkin_ref,
                 wvin_ref, wdect_ref, bdec_ref, ebias_ref, wkout_ref,
                 mq_ref, pos_ref,
                 wtd_ref,
                 h_sc, sems, alpha_sc, count_sc, v_sc, vdot_sc, mk_sc,
                 m_sc, z_sc, cv_sc, acc_sc):
    i = pl.program_id(0)

    @pl.when(i == 0)
    def _prep():
        for b in range(B):
            pltpu.make_async_copy(
                emb_ref.at[pl.ds(tok_ref[b], 1), :],
                h_sc.at[pl.ds(b, 1), :], sems.at[b % 4]).start()

        col_iota = jax.lax.broadcasted_iota(jnp.int32, (N_in, N), 1)
        P = (col_iota == pos_ref[...]).astype(jnp.float32)
        in_ids = jax.lax.dot_general(P, colidt_ref[...],
                                     (((1,), (1,)), ((), ())))
        keys = jax.lax.dot_general(in_ids, wkin_ref[...],
                                   (((1,), (0,)), ((), ())))
        alpha_sc[...] = jax.nn.sigmoid(
            jax.lax.dot_general(wdect_ref[...], colidt_ref[...],
                                (((1,), (0,)), ((), ()))) + bdec_ref[0, 0])
        mk = jax.lax.dot_general(mq_ref[...], wkout_ref[...],
                                 (((1,), (1,)), ((), ())))
        mk_sc[...] = mk

        for b in range(B):
            pltpu.make_async_copy(
                emb_ref.at[pl.ds(tok_ref[b], 1), :],
                h_sc.at[pl.ds(b, 1), :], sems.at[b % 4]).wait()
        h = h_sc[...]

        j_iota = jax.lax.broadcasted_iota(jnp.int32, (B, N_in), 1)
        counts_in = jnp.zeros((B, N_in), jnp.float32)
        for hd in range(H):
            wqt_h = wqt_ref[hd * Dq:(hd + 1) * Dq, :]
            a_h = jax.lax.dot_general(wqt_h, keys,
                                      (((0,), (1,)), ((), ())))
            sc_h = jax.lax.dot_general(h, a_h, (((1,), (0,)), ((), ())))
            sc_h = sc_h * (1.0 / 8.0) + ebias_ref[hd:hd + 1, :]
            mx = jnp.max(sc_h, axis=1, keepdims=True)
            idx = jnp.min(jnp.where(sc_h == mx, j_iota, N_in),
                          axis=1, keepdims=True)
            counts_in = counts_in + (j_iota == idx).astype(jnp.float32)
        count_sc[...] = jax.lax.dot_general(counts_in, P,
                                            (((1,), (0,)), ((), ())))
        v = jax.lax.dot_general(h, wvin_ref[...], (((1,), (0,)), ((), ())))
        v_sc[...] = v
        vdot_sc[...] = jax.lax.dot_general(v, mk_sc[...],
                                           (((1,), (1,)), ((), ())))
        m_sc[...] = jnp.full((B, 1), -1e30, jnp.float32)
        z_sc[...] = jnp.zeros((B, 1), jnp.float32)
        cv_sc[...] = jnp.zeros((B, 1), jnp.float32)
        acc_sc[...] = jnp.zeros((B, D_s), jnp.float32)

    s_blk = s_ref[...]
    sdot = jax.lax.dot_general(
        s_blk.reshape(B * NB, D_s), mk_sc[...],
        (((1,), (1,)), ((), ()))).reshape(B, NB, 1)[:, :, 0]
    alpha = alpha_sc[0:1, pl.ds(i * NB, NB)]
    cnt = count_sc[:, pl.ds(i * NB, NB)]
    logit = (alpha * sdot + cnt * vdot_sc[...]) * (1.0 / 16.0)
    m_old = m_sc[...]
    m_new = jnp.maximum(m_old, jnp.max(logit, axis=1, keepdims=True))
    corr = jnp.exp(m_old - m_new)
    p = jnp.exp(logit - m_new)
    m_sc[...] = m_new
    z_sc[...] = z_sc[...] * corr + jnp.sum(p, axis=1, keepdims=True)
    cv_sc[...] = cv_sc[...] * corr + jnp.sum(p * cnt, axis=1, keepdims=True)
    pa = p * alpha
    contrib = jax.lax.dot_general(pa, s_blk,
                                  (((1,), (1,)), ((0,), (0,))))
    acc_sc[...] = acc_sc[...] * corr + contrib

    @pl.when(i == (N // NB) - 1)
    def _fin():
        wtd_ref[...] = (acc_sc[...] + cv_sc[...] * v_sc[...]) / z_sc[...]


def _stream(token_id, tok_emb, s, WqT, col_idT, Wk_in, Wv_in, w_decayT,
            b_decay, input_E_bias, Wk_out, motor_query, input_positions):
    cst = lambda i, tok: (0, 0)
    return pl.pallas_call(
        _stream_body,
        grid_spec=pltpu.PrefetchScalarGridSpec(
            num_scalar_prefetch=1,
            grid=(N // NB,),
            in_specs=[
                pl.BlockSpec(memory_space=pltpu.MemorySpace.HBM),
                pl.BlockSpec((B, NB, D_s), lambda i, tok: (0, i, 0)),
                pl.BlockSpec((H * Dq, D_s), cst),
                pl.BlockSpec((D_id, N), cst),
                pl.BlockSpec((D_id, Dq), cst),
                pl.BlockSpec((D_s, D_s), cst),
                pl.BlockSpec((1, D_id), cst),
                pl.BlockSpec((1, 1), cst),
                pl.BlockSpec((H, N_in), cst),
                pl.BlockSpec((D_s, D_s), cst),
                pl.BlockSpec((1, D_s), cst),
                pl.BlockSpec((N_in, 1), cst),
            ],
            out_specs=pl.BlockSpec((B, D_s), cst),
            scratch_shapes=[
                pltpu.VMEM((B, D_s), jnp.float32),
                pltpu.SemaphoreType.DMA((4,)),
                pltpu.VMEM((1, N), jnp.float32),
                pltpu.VMEM((B, N), jnp.float32),
                pltpu.VMEM((B, D_s), jnp.float32),
                pltpu.VMEM((B, 1), jnp.float32),
                pltpu.VMEM((1, D_s), jnp.float32),
                pltpu.VMEM((B, 1), jnp.float32),
                pltpu.VMEM((B, 1), jnp.float32),
                pltpu.VMEM((B, 1), jnp.float32),
                pltpu.VMEM((B, D_s), jnp.float32),
            ],
        ),
        out_shape=jax.ShapeDtypeStruct((B, D_s), jnp.float32),
    )(token_id, tok_emb, s, WqT, col_idT, Wk_in, Wv_in, w_decayT,
      b_decay.reshape(1, 1), input_E_bias, Wk_out,
      motor_query.reshape(1, D_s), input_positions.reshape(N_in, 1))


def _logits_body(wtd_ref, wvout_ref, emb_ref, out_ref, motor_sc):
    @pl.when(pl.program_id(0) == 0)
    def _motor():
        motor = jax.lax.dot_general(wtd_ref[...], wvout_ref[...],
                                    (((1,), (0,)), ((), ())))
        ms = jnp.mean(motor * motor, axis=-1, keepdims=True)
        motor_sc[...] = motor * jax.lax.rsqrt(ms + 1e-6)

    out_ref[...] = jax.lax.dot_general(motor_sc[...], emb_ref[...],
                                       (((1,), (1,)), ((), ())))


def _logits(weighted, Wv_out, tok_emb):
    return pl.pallas_call(
        _logits_body,
        grid=(V // VB,),
        in_specs=[
            pl.BlockSpec((B, D_s), lambda i: (0, 0)),
            pl.BlockSpec((D_s, D_s), lambda i: (0, 0)),
            pl.BlockSpec((VB, D_s), lambda i: (i, 0)),
        ],
        out_specs=pl.BlockSpec((B, VB), lambda i: (0, i)),
        out_shape=jax.ShapeDtypeStruct((B, V), jnp.float32),
        scratch_shapes=[pltpu.VMEM((B, D_s), jnp.float32)],
    )(weighted, Wv_out, tok_emb)


def kernel(token_id, s, tok_emb, Wq, col_id, Wk_in, Wv_in, w_decay, b_decay,
           input_E_bias, Wk_out, Wv_out, motor_query, input_positions):
    weighted = _stream(token_id, tok_emb, s, Wq.T, col_id.T, Wk_in, Wv_in,
                       w_decay.T, b_decay, input_E_bias, Wk_out,
                       motor_query, input_positions)
    return _logits(weighted, Wv_out, tok_emb)


# speedup vs baseline: 1.2969x; 1.0860x over previous
import jax
import jax.numpy as jnp
from jax.experimental import pallas as pl
from jax.experimental.pallas import tpu as pltpu

B, N, D_s, D_id, H, Dq, N_in, V = 128, 1024, 256, 64, 4, 64, 256, 32768

CB = 16
NC = B // CB
VB = 4096


def _schunk_copy(s_ref, buf, sems, c):
    return pltpu.make_async_copy(
        s_ref.at[pl.ds(c * CB, CB), :, :], buf, sems.at[c % 2])


def _stream_body(tok_ref, emb_ref, s_ref, wqt_ref, colidt_ref, wkin_ref,
                 wvin_ref, wdect_ref, bdec_ref, ebias_ref, wkout_ref,
                 mq_ref, pos_ref,
                 wtd_ref,
                 h_sc, gsems, sbuf0, sbuf1, ssems):
    sbufs = (sbuf0, sbuf1)
    for b in range(B):
        pltpu.make_async_copy(
            emb_ref.at[pl.ds(tok_ref[b], 1), :],
            h_sc.at[pl.ds(b, 1), :], gsems.at[b % 4]).start()
    _schunk_copy(s_ref, sbuf0, ssems, 0).start()
    _schunk_copy(s_ref, sbuf1, ssems, 1).start()

    col_iota = jax.lax.broadcasted_iota(jnp.int32, (N_in, N), 1)
    P = (col_iota == pos_ref[...]).astype(jnp.float32)
    in_ids = jax.lax.dot_general(P, colidt_ref[...],
                                 (((1,), (1,)), ((), ())))
    keys = jax.lax.dot_general(in_ids, wkin_ref[...],
                               (((1,), (0,)), ((), ())))
    alpha = jax.nn.sigmoid(
        jax.lax.dot_general(wdect_ref[...], colidt_ref[...],
                            (((1,), (0,)), ((), ()))) + bdec_ref[0, 0])
    mk = jax.lax.dot_general(mq_ref[...], wkout_ref[...],
                             (((1,), (1,)), ((), ())))

    for b in range(B):
        pltpu.make_async_copy(
            emb_ref.at[pl.ds(tok_ref[b], 1), :],
            h_sc.at[pl.ds(b, 1), :], gsems.at[b % 4]).wait()
    h = h_sc[...]

    j_iota = jax.lax.broadcasted_iota(jnp.int32, (B, N_in), 1)
    counts_in = jnp.zeros((B, N_in), jnp.float32)
    for hd in range(H):
        wqt_h = wqt_ref[hd * Dq:(hd + 1) * Dq, :]
        a_h = jax.lax.dot_general(wqt_h, keys,
                                  (((0,), (1,)), ((), ())))
        sc_h = jax.lax.dot_general(h, a_h, (((1,), (0,)), ((), ())))
        sc_h = sc_h * (1.0 / 8.0) + ebias_ref[hd:hd + 1, :]
        mx = jnp.max(sc_h, axis=1, keepdims=True)
        idx = jnp.min(jnp.where(sc_h == mx, j_iota, N_in),
                      axis=1, keepdims=True)
        counts_in = counts_in + (j_iota == idx).astype(jnp.float32)
    count = jax.lax.dot_general(counts_in, P,
                                (((1,), (0,)), ((), ())))
    v = jax.lax.dot_general(h, wvin_ref[...], (((1,), (0,)), ((), ())))
    vdot = jax.lax.dot_general(v, mk, (((1,), (1,)), ((), ())))

    for c in range(NC):
        _schunk_copy(s_ref, sbufs[c % 2], ssems, c).wait()
        s_blk = sbufs[c % 2][...]
        if c + 2 < NC:
            _schunk_copy(s_ref, sbufs[c % 2], ssems, c + 2).start()
        lo, hi = c * CB, (c + 1) * CB
        sdot = jax.lax.dot_general(
            s_blk.reshape(CB * N, D_s), mk,
            (((1,), (1,)), ((), ()))).reshape(CB, N, 1)[:, :, 0]
        cnt = count[lo:hi, :]
        logit = (alpha * sdot + cnt * vdot[lo:hi, :]) * (1.0 / 16.0)
        m = jnp.max(logit, axis=1, keepdims=True)
        p = jnp.exp(logit - m)
        z = jnp.sum(p, axis=1, keepdims=True)
        cv = jnp.sum(p * cnt, axis=1, keepdims=True)
        pa = p * alpha
        acc = jax.lax.dot_general(pa, s_blk,
                                  (((1,), (1,)), ((0,), (0,))))
        wtd_ref[lo:hi, :] = (acc + cv * v[lo:hi, :]) / z


def _stream(token_id, tok_emb, s, WqT, col_idT, Wk_in, Wv_in, w_decayT,
            b_decay, input_E_bias, Wk_out, motor_query, input_positions):
    cst = lambda i, tok: (0, 0)
    return pl.pallas_call(
        _stream_body,
        grid_spec=pltpu.PrefetchScalarGridSpec(
            num_scalar_prefetch=1,
            grid=(1,),
            in_specs=[
                pl.BlockSpec(memory_space=pltpu.MemorySpace.HBM),
                pl.BlockSpec(memory_space=pltpu.MemorySpace.HBM),
                pl.BlockSpec((H * Dq, D_s), cst),
                pl.BlockSpec((D_id, N), cst),
                pl.BlockSpec((D_id, Dq), cst),
                pl.BlockSpec((D_s, D_s), cst),
                pl.BlockSpec((1, D_id), cst),
                pl.BlockSpec((1, 1), cst),
                pl.BlockSpec((H, N_in), cst),
                pl.BlockSpec((D_s, D_s), cst),
                pl.BlockSpec((1, D_s), cst),
                pl.BlockSpec((N_in, 1), cst),
            ],
            out_specs=pl.BlockSpec((B, D_s), cst),
            scratch_shapes=[
                pltpu.VMEM((B, D_s), jnp.float32),
                pltpu.SemaphoreType.DMA((4,)),
                pltpu.VMEM((CB, N, D_s), jnp.float32),
                pltpu.VMEM((CB, N, D_s), jnp.float32),
                pltpu.SemaphoreType.DMA((2,)),
            ],
        ),
        out_shape=jax.ShapeDtypeStruct((B, D_s), jnp.float32),
    )(token_id, tok_emb, s, WqT, col_idT, Wk_in, Wv_in, w_decayT,
      b_decay.reshape(1, 1), input_E_bias, Wk_out,
      motor_query.reshape(1, D_s), input_positions.reshape(N_in, 1))


def _logits_body(wtd_ref, wvout_ref, emb_ref, out_ref, motor_sc):
    @pl.when(pl.program_id(0) == 0)
    def _motor():
        motor = jax.lax.dot_general(wtd_ref[...], wvout_ref[...],
                                    (((1,), (0,)), ((), ())))
        ms = jnp.mean(motor * motor, axis=-1, keepdims=True)
        motor_sc[...] = motor * jax.lax.rsqrt(ms + 1e-6)

    out_ref[...] = jax.lax.dot_general(motor_sc[...], emb_ref[...],
                                       (((1,), (1,)), ((), ())))


def _logits(weighted, Wv_out, tok_emb):
    return pl.pallas_call(
        _logits_body,
        grid=(V // VB,),
        in_specs=[
            pl.BlockSpec((B, D_s), lambda i: (0, 0)),
            pl.BlockSpec((D_s, D_s), lambda i: (0, 0)),
            pl.BlockSpec((VB, D_s), lambda i: (i, 0)),
        ],
        out_specs=pl.BlockSpec((B, VB), lambda i: (0, i)),
        out_shape=jax.ShapeDtypeStruct((B, V), jnp.float32),
        scratch_shapes=[pltpu.VMEM((B, D_s), jnp.float32)],
    )(weighted, Wv_out, tok_emb)


def kernel(token_id, s, tok_emb, Wq, col_id, Wk_in, Wv_in, w_decay, b_decay,
           input_E_bias, Wk_out, Wv_out, motor_query, input_positions):
    weighted = _stream(token_id, tok_emb, s, Wq.T, col_id.T, Wk_in, Wv_in,
                       w_decay.T, b_decay, input_E_bias, Wk_out,
                       motor_query, input_positions)
    return _logits(weighted, Wv_out, tok_emb)


# speedup vs baseline: 1.3634x; 1.0513x over previous
import jax
import jax.numpy as jnp
from jax.experimental import pallas as pl
from jax.experimental.pallas import tpu as pltpu

B, N, D_s, D_id, H, Dq, N_in, V = 128, 1024, 256, 64, 4, 64, 256, 32768

CB = 16
NC = B // CB
VB = 4096


def _schunk_copy(s_ref, buf, sems, c):
    return pltpu.make_async_copy(
        s_ref.at[pl.ds(c * CB, CB), :, :], buf, sems.at[c % 2])


def _stream_body(tok_ref, emb_ref, s_ref, wqt_ref, colidt_ref, wkin_ref,
                 wvin_ref, wdect_ref, bdec_ref, ebias_ref, wkout_ref,
                 mq_ref, pos_ref, wvout_ref,
                 out_ref,
                 h_sc, gsems, sbuf0, sbuf1, ssems,
                 ebuf0, ebuf1, esems, obuf0, obuf1, osems, wtd_sc):
    sbufs = (sbuf0, sbuf1)
    ebufs = (ebuf0, ebuf1)
    obufs = (obuf0, obuf1)

    def _etile_copy(t, buf):
        return pltpu.make_async_copy(
            emb_ref.at[pl.ds(t * VB, VB), :], buf, esems.at[t % 2])

    def _otile_copy(t, buf):
        return pltpu.make_async_copy(
            buf, out_ref.at[:, pl.ds(t * VB, VB)], osems.at[t % 2])
    for b in range(B):
        pltpu.make_async_copy(
            emb_ref.at[pl.ds(tok_ref[b], 1), :],
            h_sc.at[pl.ds(b, 1), :], gsems.at[b % 4]).start()
    _schunk_copy(s_ref, sbuf0, ssems, 0).start()
    _schunk_copy(s_ref, sbuf1, ssems, 1).start()

    col_iota = jax.lax.broadcasted_iota(jnp.int32, (N_in, N), 1)
    P = (col_iota == pos_ref[...]).astype(jnp.float32)
    in_ids = jax.lax.dot_general(P, colidt_ref[...],
                                 (((1,), (1,)), ((), ())))
    keys = jax.lax.dot_general(in_ids, wkin_ref[...],
                               (((1,), (0,)), ((), ())))
    alpha = jax.nn.sigmoid(
        jax.lax.dot_general(wdect_ref[...], colidt_ref[...],
                            (((1,), (0,)), ((), ()))) + bdec_ref[0, 0])
    mk = jax.lax.dot_general(mq_ref[...], wkout_ref[...],
                             (((1,), (1,)), ((), ())))

    for b in range(B):
        pltpu.make_async_copy(
            emb_ref.at[pl.ds(tok_ref[b], 1), :],
            h_sc.at[pl.ds(b, 1), :], gsems.at[b % 4]).wait()
    h = h_sc[...]

    j_iota = jax.lax.broadcasted_iota(jnp.int32, (B, N_in), 1)
    counts_in = jnp.zeros((B, N_in), jnp.float32)
    for hd in range(H):
        wqt_h = wqt_ref[hd * Dq:(hd + 1) * Dq, :]
        a_h = jax.lax.dot_general(wqt_h, keys,
                                  (((0,), (1,)), ((), ())))
        sc_h = jax.lax.dot_general(h, a_h, (((1,), (0,)), ((), ())))
        sc_h = sc_h * (1.0 / 8.0) + ebias_ref[hd:hd + 1, :]
        mx = jnp.max(sc_h, axis=1, keepdims=True)
        idx = jnp.min(jnp.where(sc_h == mx, j_iota, N_in),
                      axis=1, keepdims=True)
        counts_in = counts_in + (j_iota == idx).astype(jnp.float32)
    count = jax.lax.dot_general(counts_in, P,
                                (((1,), (0,)), ((), ())))
    v = jax.lax.dot_general(h, wvin_ref[...], (((1,), (0,)), ((), ())))
    vdot = jax.lax.dot_general(v, mk, (((1,), (1,)), ((), ())))

    for c in range(NC):
        _schunk_copy(s_ref, sbufs[c % 2], ssems, c).wait()
        s_blk = sbufs[c % 2][...]
        if c + 2 < NC:
            _schunk_copy(s_ref, sbufs[c % 2], ssems, c + 2).start()
        if c >= NC - 2:
            _etile_copy(c - (NC - 2), ebufs[c - (NC - 2)]).start()
        lo, hi = c * CB, (c + 1) * CB
        sdot = jax.lax.dot_general(
            s_blk.reshape(CB * N, D_s), mk,
            (((1,), (1,)), ((), ()))).reshape(CB, N, 1)[:, :, 0]
        cnt = count[lo:hi, :]
        logit = (alpha * sdot + cnt * vdot[lo:hi, :]) * (1.0 / 16.0)
        m = jnp.max(logit, axis=1, keepdims=True)
        p = jnp.exp(logit - m)
        z = jnp.sum(p, axis=1, keepdims=True)
        cv = jnp.sum(p * cnt, axis=1, keepdims=True)
        pa = p * alpha
        acc = jax.lax.dot_general(pa, s_blk,
                                  (((1,), (1,)), ((0,), (0,))))
        wtd_sc[lo:hi, :] = (acc + cv * v[lo:hi, :]) / z


    motor = jax.lax.dot_general(wtd_sc[...], wvout_ref[...],
                                (((1,), (0,)), ((), ())))
    ms = jnp.mean(motor * motor, axis=-1, keepdims=True)
    motor = motor * jax.lax.rsqrt(ms + 1e-6)
    NT = V // VB
    for t in range(NT):
        _etile_copy(t, ebufs[t % 2]).wait()
        tile = jax.lax.dot_general(motor, ebufs[t % 2][...],
                                   (((1,), (1,)), ((), ())))
        if t + 2 < NT:
            _etile_copy(t + 2, ebufs[t % 2]).start()
        if t >= 2:
            _otile_copy(t - 2, obufs[t % 2]).wait()
        obufs[t % 2][...] = tile
        _otile_copy(t, obufs[t % 2]).start()
    _otile_copy(NT - 2, obufs[(NT - 2) % 2]).wait()
    _otile_copy(NT - 1, obufs[(NT - 1) % 2]).wait()


def _stream(token_id, tok_emb, s, WqT, col_idT, Wk_in, Wv_in, w_decayT,
            b_decay, input_E_bias, Wk_out, motor_query, input_positions,
            Wv_out):
    cst = lambda i, tok: (0, 0)
    return pl.pallas_call(
        _stream_body,
        grid_spec=pltpu.PrefetchScalarGridSpec(
            num_scalar_prefetch=1,
            grid=(1,),
            in_specs=[
                pl.BlockSpec(memory_space=pltpu.MemorySpace.HBM),
                pl.BlockSpec(memory_space=pltpu.MemorySpace.HBM),
                pl.BlockSpec((H * Dq, D_s), cst),
                pl.BlockSpec((D_id, N), cst),
                pl.BlockSpec((D_id, Dq), cst),
                pl.BlockSpec((D_s, D_s), cst),
                pl.BlockSpec((1, D_id), cst),
                pl.BlockSpec((1, 1), cst),
                pl.BlockSpec((H, N_in), cst),
                pl.BlockSpec((D_s, D_s), cst),
                pl.BlockSpec((1, D_s), cst),
                pl.BlockSpec((N_in, 1), cst),
                pl.BlockSpec((D_s, D_s), cst),
            ],
            out_specs=pl.BlockSpec(memory_space=pltpu.MemorySpace.HBM),
            scratch_shapes=[
                pltpu.VMEM((B, D_s), jnp.float32),
                pltpu.SemaphoreType.DMA((4,)),
                pltpu.VMEM((CB, N, D_s), jnp.float32),
                pltpu.VMEM((CB, N, D_s), jnp.float32),
                pltpu.SemaphoreType.DMA((2,)),
                pltpu.VMEM((VB, D_s), jnp.float32),
                pltpu.VMEM((VB, D_s), jnp.float32),
                pltpu.SemaphoreType.DMA((2,)),
                pltpu.VMEM((B, VB), jnp.float32),
                pltpu.VMEM((B, VB), jnp.float32),
                pltpu.SemaphoreType.DMA((2,)),
                pltpu.VMEM((B, D_s), jnp.float32),
            ],
        ),
        out_shape=jax.ShapeDtypeStruct((B, V), jnp.float32),
    )(token_id, tok_emb, s, WqT, col_idT, Wk_in, Wv_in, w_decayT,
      b_decay.reshape(1, 1), input_E_bias, Wk_out,
      motor_query.reshape(1, D_s), input_positions.reshape(N_in, 1), Wv_out)


def kernel(token_id, s, tok_emb, Wq, col_id, Wk_in, Wv_in, w_decay, b_decay,
           input_E_bias, Wk_out, Wv_out, motor_query, input_positions):
    return _stream(token_id, tok_emb, s, Wq.T, col_id.T, Wk_in, Wv_in,
                   w_decay.T, b_decay, input_E_bias, Wk_out,
                   motor_query, input_positions, Wv_out)
